# baseline (speedup 1.0000x reference)
import jax
import jax.numpy as jnp
from jax.experimental import pallas as pl
from jax.experimental.pallas import tpu as pltpu

_E = 8
_R = 16
_ER = _E * _R
_SCALING = 32.0 / 16.0
_TT = 2048
_FT = 128

_CT = (((1,), (1,)), ((), ()))


def _dot_t(a, b):
    return jax.lax.dot_general(a, b, _CT, preferred_element_type=jnp.float32)


def _moe_kernel(x_ref, gw_ref, wg_ref, wu_ref, wd_ref,
                ag_ref, bg_ref, au_ref, bu_ref, ad_ref, bd_ref,
                o_ref, w_scr, m_scr, xg_scr, xu_scr, p_scr):
    f = pl.program_id(0)
    nf = pl.num_programs(0)

    @pl.when(f == 0)
    def _init():
        x = x_ref[...]
        logits = _dot_t(x, gw_ref[...])
        probs = jax.nn.softmax(logits, axis=-1)
        eidx = jax.lax.broadcasted_iota(jnp.int32, probs.shape, 1)
        m1 = jnp.max(probs, axis=-1, keepdims=True)
        i1 = jnp.min(jnp.where(probs == m1, eidx, _E), axis=-1, keepdims=True)
        pm = jnp.where(eidx == i1, -1.0, probs)
        m2 = jnp.max(pm, axis=-1, keepdims=True)
        i2 = jnp.min(jnp.where(pm == m2, eidx, _E), axis=-1, keepdims=True)
        denom = m1 + m2
        zero6 = jnp.zeros((_TT, 6), jnp.float32)
        w_scr[...] = jnp.concatenate([m1 / denom, m2 / denom, zero6], axis=1)
        lane_e = jax.lax.broadcasted_iota(jnp.int32, (_TT, _ER), 1) // _R
        mask1 = jnp.where(lane_e == i1, _SCALING, 0.0)
        mask2 = jnp.where(lane_e == i2, _SCALING, 0.0)
        m_scr[0] = mask1
        m_scr[1] = mask2
        xag = _dot_t(x, ag_ref[...])
        xau = _dot_t(x, au_ref[...])
        xg_scr[0] = xag * mask1
        xg_scr[1] = xag * mask2
        xu_scr[0] = xau * mask1
        xu_scr[1] = xau * mask2
        p_scr[...] = jnp.zeros_like(p_scr)
        o_ref[...] = jnp.zeros_like(o_ref)

    g0 = _dot_t(x_ref[...], wg_ref[...])
    u0 = _dot_t(x_ref[...], wu_ref[...])
    w = w_scr[...]
    mix = jnp.zeros_like(g0)
    for s in range(2):
        lg = jnp.dot(xg_scr[s], bg_ref[...],
                     preferred_element_type=jnp.float32)
        lu = jnp.dot(xu_scr[s], bu_ref[...],
                     preferred_element_type=jnp.float32)
        inter = jax.nn.silu(g0 + lg) * (u0 + lu)
        wi = w[:, s:s + 1] * inter
        mix = mix + wi
        p_scr[s] += _dot_t(wi, ad_ref[...])
    o_ref[...] += _dot_t(mix, wd_ref[...])

    @pl.when(f == nf - 1)
    def _down_lora():
        p = p_scr[0] * m_scr[0] + p_scr[1] * m_scr[1]
        o_ref[...] += jnp.dot(
            p, bd_ref[...], preferred_element_type=jnp.float32)


def kernel(hidden_states, gate_w, w_gate, w_up, w_down,
           lora_a_gate, lora_b_gate, lora_a_up, lora_b_up,
           lora_a_down, lora_b_down):
    b, s, d = hidden_states.shape
    x = hidden_states.reshape(-1, d)
    t = x.shape[0]
    ff = w_gate.shape[0]
    bg = jnp.swapaxes(lora_b_gate, 1, 2).reshape(_ER, ff)
    bu = jnp.swapaxes(lora_b_up, 1, 2).reshape(_ER, ff)
    ag = lora_a_gate.reshape(_ER, d)
    au = lora_a_up.reshape(_ER, d)
    ad = lora_a_down.reshape(_ER, ff)
    bd = jnp.swapaxes(lora_b_down, 1, 2).reshape(_ER, d)

    grid = (ff // _FT,)
    out = pl.pallas_call(
        _moe_kernel,
        grid=grid,
        in_specs=[
            pl.BlockSpec((_TT, d), lambda f: (0, 0)),
            pl.BlockSpec((_E, d), lambda f: (0, 0)),
            pl.BlockSpec((_FT, d), lambda f: (f, 0)),
            pl.BlockSpec((_FT, d), lambda f: (f, 0)),
            pl.BlockSpec((d, _FT), lambda f: (0, f)),
            pl.BlockSpec((_ER, d), lambda f: (0, 0)),
            pl.BlockSpec((_ER, _FT), lambda f: (0, f)),
            pl.BlockSpec((_ER, d), lambda f: (0, 0)),
            pl.BlockSpec((_ER, _FT), lambda f: (0, f)),
            pl.BlockSpec((_ER, _FT), lambda f: (0, f)),
            pl.BlockSpec((_ER, d), lambda f: (0, 0)),
        ],
        out_specs=pl.BlockSpec((_TT, d), lambda f: (0, 0)),
        out_shape=jax.ShapeDtypeStruct((t, d), jnp.float32),
        scratch_shapes=[
            pltpu.VMEM((_TT, _E), jnp.float32),
            pltpu.VMEM((2, _TT, _ER), jnp.float32),
            pltpu.VMEM((2, _TT, _ER), jnp.float32),
            pltpu.VMEM((2, _TT, _ER), jnp.float32),
            pltpu.VMEM((2, _TT, _ER), jnp.float32),
        ],
        compiler_params=pltpu.CompilerParams(
            dimension_semantics=("arbitrary",)),
    )(x, gate_w, w_gate, w_up, w_down, ag, bg, au, bu, ad, bd)
    return out.reshape(b, s, d)


# speedup vs baseline: 1.5304x; 1.5304x over previous
import jax
import jax.numpy as jnp
from jax.experimental import pallas as pl
from jax.experimental.pallas import tpu as pltpu

_E = 8
_R = 16
_ER = _E * _R
_SCALING = 32.0 / 16.0
_TT = 2048
_FT = 256

_CT = (((1,), (1,)), ((), ()))


def _dot_t(a, b):
    return jax.lax.dot_general(a, b, _CT, preferred_element_type=jnp.float32)


def _moe_kernel(x_ref, gw_ref, wg_ref, wu_ref, wd_ref,
                ag_ref, bg_ref, au_ref, bu_ref, ad_ref, bd_ref,
                o_ref, w_scr, m_scr, xg_scr, xu_scr, p_scr):
    f = pl.program_id(0)
    nf = pl.num_programs(0)

    @pl.when(f == 0)
    def _init():
        x = x_ref[...]
        logits = _dot_t(x, gw_ref[...])
        probs = jax.nn.softmax(logits, axis=-1)
        eidx = jax.lax.broadcasted_iota(jnp.int32, probs.shape, 1)
        m1 = jnp.max(probs, axis=-1, keepdims=True)
        i1 = jnp.min(jnp.where(probs == m1, eidx, _E), axis=-1, keepdims=True)
        pm = jnp.where(eidx == i1, -1.0, probs)
        m2 = jnp.max(pm, axis=-1, keepdims=True)
        i2 = jnp.min(jnp.where(pm == m2, eidx, _E), axis=-1, keepdims=True)
        denom = m1 + m2
        zero6 = jnp.zeros((_TT, 6), jnp.float32)
        w_scr[...] = jnp.concatenate([m1 / denom, m2 / denom, zero6], axis=1)
        lane_e = jax.lax.broadcasted_iota(jnp.int32, (_TT, _ER), 1) // _R
        mask1 = jnp.where(lane_e == i1, _SCALING, 0.0)
        mask2 = jnp.where(lane_e == i2, _SCALING, 0.0)
        m_scr[0] = mask1
        m_scr[1] = mask2
        xag = _dot_t(x, ag_ref[...])
        xau = _dot_t(x, au_ref[...])
        xg_scr[0] = xag * mask1
        xg_scr[1] = xag * mask2
        xu_scr[0] = xau * mask1
        xu_scr[1] = xau * mask2
        p_scr[...] = jnp.zeros_like(p_scr)
        o_ref[...] = jnp.zeros_like(o_ref)

    g0 = _dot_t(x_ref[...], wg_ref[...])
    u0 = _dot_t(x_ref[...], wu_ref[...])
    w = w_scr[...]
    lg0 = jnp.dot(xg_scr[0], bg_ref[...], preferred_element_type=jnp.float32)
    lu0 = jnp.dot(xu_scr[0], bu_ref[...], preferred_element_type=jnp.float32)
    lg1 = jnp.dot(xg_scr[1], bg_ref[...], preferred_element_type=jnp.float32)
    lu1 = jnp.dot(xu_scr[1], bu_ref[...], preferred_element_type=jnp.float32)
    wi0 = w[:, 0:1] * (jax.nn.silu(g0 + lg0) * (u0 + lu0))
    wi1 = w[:, 1:2] * (jax.nn.silu(g0 + lg1) * (u0 + lu1))
    mix = wi0 + wi1
    p_scr[0] += _dot_t(wi0, ad_ref[...])
    p_scr[1] += _dot_t(wi1, ad_ref[...])
    o_ref[...] += _dot_t(mix, wd_ref[...])

    @pl.when(f == nf - 1)
    def _down_lora():
        p = p_scr[0] * m_scr[0] + p_scr[1] * m_scr[1]
        o_ref[...] += jnp.dot(
            p, bd_ref[...], preferred_element_type=jnp.float32)


def kernel(hidden_states, gate_w, w_gate, w_up, w_down,
           lora_a_gate, lora_b_gate, lora_a_up, lora_b_up,
           lora_a_down, lora_b_down):
    b, s, d = hidden_states.shape
    x = hidden_states.reshape(-1, d)
    t = x.shape[0]
    ff = w_gate.shape[0]
    bg = jnp.swapaxes(lora_b_gate, 1, 2).reshape(_ER, ff)
    bu = jnp.swapaxes(lora_b_up, 1, 2).reshape(_ER, ff)
    ag = lora_a_gate.reshape(_ER, d)
    au = lora_a_up.reshape(_ER, d)
    ad = lora_a_down.reshape(_ER, ff)
    bd = jnp.swapaxes(lora_b_down, 1, 2).reshape(_ER, d)

    grid = (ff // _FT,)
    out = pl.pallas_call(
        _moe_kernel,
        grid=grid,
        in_specs=[
            pl.BlockSpec((_TT, d), lambda f: (0, 0)),
            pl.BlockSpec((_E, d), lambda f: (0, 0)),
            pl.BlockSpec((_FT, d), lambda f: (f, 0)),
            pl.BlockSpec((_FT, d), lambda f: (f, 0)),
            pl.BlockSpec((d, _FT), lambda f: (0, f)),
            pl.BlockSpec((_ER, d), lambda f: (0, 0)),
            pl.BlockSpec((_ER, _FT), lambda f: (0, f)),
            pl.BlockSpec((_ER, d), lambda f: (0, 0)),
            pl.BlockSpec((_ER, _FT), lambda f: (0, f)),
            pl.BlockSpec((_ER, _FT), lambda f: (0, f)),
            pl.BlockSpec((_ER, d), lambda f: (0, 0)),
        ],
        out_specs=pl.BlockSpec((_TT, d), lambda f: (0, 0)),
        out_shape=jax.ShapeDtypeStruct((t, d), jnp.float32),
        scratch_shapes=[
            pltpu.VMEM((_TT, _E), jnp.float32),
            pltpu.VMEM((2, _TT, _ER), jnp.float32),
            pltpu.VMEM((2, _TT, _ER), jnp.float32),
            pltpu.VMEM((2, _TT, _ER), jnp.float32),
            pltpu.VMEM((2, _TT, _ER), jnp.float32),
        ],
        compiler_params=pltpu.CompilerParams(
            dimension_semantics=("arbitrary",)),
    )(x, gate_w, w_gate, w_up, w_down, ag, bg, au, bu, ad, bd)
    return out.reshape(b, s, d)
